# Initial kernel scaffold; baseline (speedup 1.0000x reference)
import jax
import jax.numpy as jnp
from jax.experimental import pallas as pl


def kernel(x, edge_index, global_features, W_emb, b_emb, Wc0, bc0, Wc1, bc1, Wc2, bc2, Wg, bg, W1, b1, W2, b2):
    raise NotImplementedError("write your pallas kernel here")


# speedup vs baseline: 11.3523x; 11.3523x over previous
import functools

import jax
import jax.numpy as jnp
from jax import lax
from jax.experimental import pallas as pl
from jax.experimental.pallas import tpu as pltpu
from jax.experimental.pallas import tpu_sc as plsc

N = 10000
E = 320000
D = 128
HF = 128
G = 32
OUT = 64

NC = 2
NS = 16
NW = NC * NS
EPT = E // NW
CHUNK = 128
NFULL = EPT // CHUNK
TAIL = EPT - NFULL * CHUNK
ROWS_PT = 624
ROWS_TL = N - NS * ROWS_PT
RCHUNK = 104
@functools.cache
def _mesh():
    return plsc.VectorSubcoreMesh(core_axis_name="c", subcore_axis_name="s",
                                  num_cores=NC, num_subcores=NS)


def _gs_body(g_hbm, src_hbm, dst_hbm, out_hbm,
             sidx_v, didx_v, rows_v, sidx_t, didx_t, rows_t,
             acc_sh, sem):
    c = lax.axis_index("c")
    s = lax.axis_index("s")
    wid = s * NC + c
    base = pl.multiple_of(wid * EPT, 8)
    r0 = pl.multiple_of(s * ROWS_PT, 8)

    for k in range(ROWS_PT // RCHUNK):
        rk = pl.multiple_of(r0 + k * RCHUNK, 8)
        pltpu.sync_copy(g_hbm.at[pl.ds(rk, RCHUNK)],
                        rows_v.at[pl.ds(0, RCHUNK)])
        pltpu.sync_copy(rows_v.at[pl.ds(0, RCHUNK)],
                        acc_sh.at[pl.ds(rk, RCHUNK)])

    @pl.when(s == 0)
    def _():
        pltpu.sync_copy(g_hbm.at[pl.ds(NS * ROWS_PT, ROWS_TL)],
                        rows_v.at[pl.ds(0, ROWS_TL)])
        pltpu.sync_copy(rows_v.at[pl.ds(0, ROWS_TL)],
                        acc_sh.at[pl.ds(NS * ROWS_PT, ROWS_TL)])

    plsc.subcore_barrier()

    def chunk(j, carry):
        off = pl.multiple_of(base + j * CHUNK, 8)
        pltpu.sync_copy(src_hbm.at[pl.ds(off, CHUNK)], sidx_v)
        pltpu.sync_copy(dst_hbm.at[pl.ds(off, CHUNK)], didx_v)
        pltpu.async_copy(g_hbm.at[sidx_v], rows_v, sem).wait()
        pltpu.sync_copy(rows_v, acc_sh.at[didx_v], add=True)
        return carry

    lax.fori_loop(0, NFULL, chunk, 0)
    off = pl.multiple_of(base + NFULL * CHUNK, 8)
    pltpu.sync_copy(src_hbm.at[pl.ds(off, TAIL)], sidx_t)
    pltpu.sync_copy(dst_hbm.at[pl.ds(off, TAIL)], didx_t)
    pltpu.async_copy(g_hbm.at[sidx_t], rows_t, sem).wait()
    pltpu.sync_copy(rows_t, acc_sh.at[didx_t], add=True)

    plsc.subcore_barrier()
    for k in range(ROWS_PT // RCHUNK):
        rk = pl.multiple_of(r0 + k * RCHUNK, 8)
        pltpu.sync_copy(acc_sh.at[pl.ds(rk, RCHUNK)],
                        rows_v.at[pl.ds(0, RCHUNK)])
        pltpu.sync_copy(rows_v.at[pl.ds(0, RCHUNK)],
                        out_hbm.at[c, pl.ds(rk, RCHUNK)])

    @pl.when(s == 0)
    def _():
        pltpu.sync_copy(acc_sh.at[pl.ds(NS * ROWS_PT, ROWS_TL)],
                        rows_v.at[pl.ds(0, ROWS_TL)])
        pltpu.sync_copy(rows_v.at[pl.ds(0, ROWS_TL)],
                        out_hbm.at[c, pl.ds(NS * ROWS_PT, ROWS_TL)])


@functools.cache
def _gs_kernel():
    return pl.kernel(
        _gs_body,
        out_type=jax.ShapeDtypeStruct((NC, N, HF), jnp.float32),
        mesh=_mesh(),
        scratch_types=[
            pltpu.VMEM((CHUNK,), jnp.int32),
            pltpu.VMEM((CHUNK,), jnp.int32),
            pltpu.VMEM((CHUNK, HF), jnp.float32),
            pltpu.VMEM((TAIL,), jnp.int32),
            pltpu.VMEM((TAIL,), jnp.int32),
            pltpu.VMEM((TAIL, HF), jnp.float32),
            pltpu.VMEM_SHARED((N, HF), jnp.float32),
            pltpu.SemaphoreType.DMA,
        ],
    )


BLK = 1000


def _mm_t(a, b):
    return lax.dot_general(a, b, (((1,), (1,)), ((), ())),
                           preferred_element_type=jnp.float32)


def _pre_body(deg_ref, x_ref, wemb_ref, bemb_ref, wc0_ref,
              g_ref, dis_ref):
    deg = deg_ref[0, :, 0] + deg_ref[1, :, 0] - 1.0
    dis = lax.rsqrt(deg)
    dis_ref[...] = dis[:, None]
    h0 = _mm_t(x_ref[...], wemb_ref[...]) + bemb_ref[...]
    g_ref[...] = _mm_t(h0, wc0_ref[...]) * dis[:, None]


def _pre_kernel(deg, x, wemb, bemb, wc0):
    return pl.pallas_call(
        _pre_body,
        grid=(N // BLK,),
        in_specs=[
            pl.BlockSpec((NC, BLK, HF), lambda i: (0, i, 0)),
            pl.BlockSpec((BLK, D), lambda i: (i, 0)),
            pl.BlockSpec((HF, D), lambda i: (0, 0)),
            pl.BlockSpec((1, HF), lambda i: (0, 0)),
            pl.BlockSpec((HF, HF), lambda i: (0, 0)),
        ],
        out_specs=[
            pl.BlockSpec((BLK, HF), lambda i: (i, 0)),
            pl.BlockSpec((BLK, 1), lambda i: (i, 0)),
        ],
        out_shape=[
            jax.ShapeDtypeStruct((N, HF), jnp.float32),
            jax.ShapeDtypeStruct((N, 1), jnp.float32),
        ],
    )(deg, x, wemb, bemb, wc0)


def _layer_body(acc_ref, g_ref, dis_ref, bc_ref, wc_ref, gout_ref):
    dis = dis_ref[...]
    h = jnp.maximum(
        dis * (acc_ref[0] + acc_ref[1] - g_ref[...]) + bc_ref[...], 0.0)
    gout_ref[...] = _mm_t(h, wc_ref[...]) * dis


def _layer_kernel(acc, g, dis, bc, wc):
    return pl.pallas_call(
        _layer_body,
        grid=(N // BLK,),
        in_specs=[
            pl.BlockSpec((NC, BLK, HF), lambda i: (0, i, 0)),
            pl.BlockSpec((BLK, HF), lambda i: (i, 0)),
            pl.BlockSpec((BLK, 1), lambda i: (i, 0)),
            pl.BlockSpec((1, HF), lambda i: (0, 0)),
            pl.BlockSpec((HF, HF), lambda i: (0, 0)),
        ],
        out_specs=pl.BlockSpec((BLK, HF), lambda i: (i, 0)),
        out_shape=jax.ShapeDtypeStruct((N, HF), jnp.float32),
    )(acc, g, dis, bc, wc)


def _head_body(acc_ref, g_ref, dis_ref, bc_ref, gf_ref,
               wg_ref, bg_ref, w1_ref, b1_ref, w2_ref, b2_ref,
               out_ref, sum_ref):
    i = pl.program_id(0)

    @pl.when(i == 0)
    def _():
        sum_ref[...] = jnp.zeros_like(sum_ref)

    h = jnp.maximum(
        dis_ref[...] * (acc_ref[0] + acc_ref[1] - g_ref[...]) + bc_ref[...],
        0.0)
    sum_ref[...] += jnp.sum(h, axis=0, keepdims=True)

    @pl.when(i == N // BLK - 1)
    def _():
        mean = sum_ref[...] * (1.0 / N)
        xg = jnp.maximum(_mm_t(gf_ref[...], wg_ref[...]) + bg_ref[...], 0.0)
        comb = jnp.concatenate([mean, xg], axis=1)
        z = jnp.maximum(_mm_t(comb, w1_ref[...]) + b1_ref[...], 0.0)
        out_ref[...] = _mm_t(z, w2_ref[...]) + b2_ref[...]


def _head_kernel(acc, g, dis, bc, gf, wg, bg, w1, b1, w2, b2):
    return pl.pallas_call(
        _head_body,
        grid=(N // BLK,),
        in_specs=[
            pl.BlockSpec((NC, BLK, HF), lambda i: (0, i, 0)),
            pl.BlockSpec((BLK, HF), lambda i: (i, 0)),
            pl.BlockSpec((BLK, 1), lambda i: (i, 0)),
            pl.BlockSpec((1, HF), lambda i: (0, 0)),
            pl.BlockSpec((1, G), lambda i: (0, 0)),
            pl.BlockSpec((HF, G), lambda i: (0, 0)),
            pl.BlockSpec((1, HF), lambda i: (0, 0)),
            pl.BlockSpec((HF, 2 * HF), lambda i: (0, 0)),
            pl.BlockSpec((1, HF), lambda i: (0, 0)),
            pl.BlockSpec((OUT, HF), lambda i: (0, 0)),
            pl.BlockSpec((1, OUT), lambda i: (0, 0)),
        ],
        out_specs=pl.BlockSpec((1, OUT), lambda i: (0, 0)),
        out_shape=jax.ShapeDtypeStruct((1, OUT), jnp.float32),
        scratch_shapes=[pltpu.VMEM((1, HF), jnp.float32)],
    )(acc, g, dis, bc, gf, wg, bg, w1, b1, w2, b2)


@jax.jit
def kernel(x, edge_index, global_features, W_emb, b_emb,
           Wc0, bc0, Wc1, bc1, Wc2, bc2, Wg, bg, W1, b1, W2, b2):
    src = edge_index[0]
    dst = edge_index[1]

    ones_n = jnp.ones((N, HF), jnp.float32)
    dacc = _gs_kernel()(ones_n, src, dst)

    g, dis = _pre_kernel(dacc, x, W_emb, b_emb.reshape(1, HF), Wc0)

    acc = _gs_kernel()(g, src, dst)
    g = _layer_kernel(acc, g, dis, bc0.reshape(1, HF), Wc1)

    acc = _gs_kernel()(g, src, dst)
    g = _layer_kernel(acc, g, dis, bc1.reshape(1, HF), Wc2)

    acc = _gs_kernel()(g, src, dst)
    out = _head_kernel(acc, g, dis, bc2.reshape(1, HF),
                       global_features.reshape(1, G),
                       Wg, bg.reshape(1, HF),
                       W1, b1.reshape(1, HF),
                       W2, b2.reshape(1, OUT))
    return out.reshape(OUT)


# speedup vs baseline: 17.3008x; 1.5240x over previous
import functools

import jax
import jax.numpy as jnp
from jax import lax
from jax.experimental import pallas as pl
from jax.experimental.pallas import tpu as pltpu
from jax.experimental.pallas import tpu_sc as plsc

N = 10000
E = 320000
D = 128
HF = 128
G = 32
OUT = 64

NC = 2
NS = 16
NW = NC * NS
EPT = E // NW
CHUNK = 128
NFULL = EPT // CHUNK
TAIL = EPT - NFULL * CHUNK
ROWS_PT = 624
ROWS_TL = N - NS * ROWS_PT
RCHUNK = 104
@functools.cache
def _mesh():
    return plsc.VectorSubcoreMesh(core_axis_name="c", subcore_axis_name="s",
                                  num_cores=NC, num_subcores=NS)


def _gs_body(g_hbm, src_hbm, dst_hbm, out_hbm,
             sidx0_v, didx0_v, rows0_v, sidx1_v, didx1_v, rows1_v,
             sidx_t, didx_t, rows_t,
             acc_sh, gsem0, gsem1, ssem0, ssem1):
    c = lax.axis_index("c")
    s = lax.axis_index("s")
    wid = s * NC + c
    base = pl.multiple_of(wid * EPT, 8)
    r0 = pl.multiple_of(s * ROWS_PT, 8)

    for k in range(ROWS_PT // RCHUNK):
        rk = pl.multiple_of(r0 + k * RCHUNK, 8)
        pltpu.sync_copy(g_hbm.at[pl.ds(rk, RCHUNK)],
                        rows0_v.at[pl.ds(0, RCHUNK)])
        pltpu.sync_copy(rows0_v.at[pl.ds(0, RCHUNK)],
                        acc_sh.at[pl.ds(rk, RCHUNK)])

    @pl.when(s == 0)
    def _():
        pltpu.sync_copy(g_hbm.at[pl.ds(NS * ROWS_PT, ROWS_TL)],
                        rows0_v.at[pl.ds(0, ROWS_TL)])
        pltpu.sync_copy(rows0_v.at[pl.ds(0, ROWS_TL)],
                        acc_sh.at[pl.ds(NS * ROWS_PT, ROWS_TL)])

    plsc.subcore_barrier()

    def pair(i, carry):
        offa = pl.multiple_of(base + (2 * i) * CHUNK, 8)
        offb = pl.multiple_of(base + (2 * i + 1) * CHUNK, 8)

        @pl.when(i > 0)
        def _():
            pltpu.make_async_copy(rows0_v, acc_sh.at[didx0_v], ssem0).wait()

        pltpu.sync_copy(src_hbm.at[pl.ds(offa, CHUNK)], sidx0_v)
        pltpu.sync_copy(dst_hbm.at[pl.ds(offa, CHUNK)], didx0_v)
        ga = pltpu.async_copy(g_hbm.at[sidx0_v], rows0_v, gsem0)

        @pl.when(i > 0)
        def _():
            pltpu.make_async_copy(rows1_v, acc_sh.at[didx1_v], ssem1).wait()

        pltpu.sync_copy(src_hbm.at[pl.ds(offb, CHUNK)], sidx1_v)
        pltpu.sync_copy(dst_hbm.at[pl.ds(offb, CHUNK)], didx1_v)
        gb = pltpu.async_copy(g_hbm.at[sidx1_v], rows1_v, gsem1)

        ga.wait()
        pltpu.async_copy(rows0_v, acc_sh.at[didx0_v], ssem0, add=True)
        gb.wait()
        pltpu.async_copy(rows1_v, acc_sh.at[didx1_v], ssem1, add=True)
        return carry

    lax.fori_loop(0, NFULL // 2, pair, 0)
    pltpu.make_async_copy(rows0_v, acc_sh.at[didx0_v], ssem0).wait()
    pltpu.make_async_copy(rows1_v, acc_sh.at[didx1_v], ssem1).wait()

    off = pl.multiple_of(base + NFULL * CHUNK, 8)
    pltpu.sync_copy(src_hbm.at[pl.ds(off, TAIL)], sidx_t)
    pltpu.sync_copy(dst_hbm.at[pl.ds(off, TAIL)], didx_t)
    pltpu.async_copy(g_hbm.at[sidx_t], rows_t, gsem0).wait()
    pltpu.sync_copy(rows_t, acc_sh.at[didx_t], add=True)

    plsc.subcore_barrier()
    for k in range(ROWS_PT // RCHUNK):
        rk = pl.multiple_of(r0 + k * RCHUNK, 8)
        pltpu.sync_copy(acc_sh.at[pl.ds(rk, RCHUNK)],
                        rows0_v.at[pl.ds(0, RCHUNK)])
        pltpu.sync_copy(rows0_v.at[pl.ds(0, RCHUNK)],
                        out_hbm.at[c, pl.ds(rk, RCHUNK)])

    @pl.when(s == 0)
    def _():
        pltpu.sync_copy(acc_sh.at[pl.ds(NS * ROWS_PT, ROWS_TL)],
                        rows0_v.at[pl.ds(0, ROWS_TL)])
        pltpu.sync_copy(rows0_v.at[pl.ds(0, ROWS_TL)],
                        out_hbm.at[c, pl.ds(NS * ROWS_PT, ROWS_TL)])


@functools.cache
def _gs_kernel():
    return pl.kernel(
        _gs_body,
        out_type=jax.ShapeDtypeStruct((NC, N, HF), jnp.float32),
        mesh=_mesh(),
        scratch_types=[
            pltpu.VMEM((CHUNK,), jnp.int32),
            pltpu.VMEM((CHUNK,), jnp.int32),
            pltpu.VMEM((CHUNK, HF), jnp.float32),
            pltpu.VMEM((CHUNK,), jnp.int32),
            pltpu.VMEM((CHUNK,), jnp.int32),
            pltpu.VMEM((CHUNK, HF), jnp.float32),
            pltpu.VMEM((TAIL,), jnp.int32),
            pltpu.VMEM((TAIL,), jnp.int32),
            pltpu.VMEM((TAIL, HF), jnp.float32),
            pltpu.VMEM_SHARED((N, HF), jnp.float32),
            pltpu.SemaphoreType.DMA,
            pltpu.SemaphoreType.DMA,
            pltpu.SemaphoreType.DMA,
            pltpu.SemaphoreType.DMA,
        ],
    )


BLK = 1000


def _mm_t(a, b):
    return lax.dot_general(a, b, (((1,), (1,)), ((), ())),
                           preferred_element_type=jnp.float32)


def _pre_body(deg_ref, x_ref, wemb_ref, bemb_ref, wc0_ref,
              g_ref, dis_ref):
    deg = deg_ref[0, :, 0] + deg_ref[1, :, 0] - 1.0
    dis = lax.rsqrt(deg)
    dis_ref[...] = dis[:, None]
    h0 = _mm_t(x_ref[...], wemb_ref[...]) + bemb_ref[...]
    g_ref[...] = _mm_t(h0, wc0_ref[...]) * dis[:, None]


def _pre_kernel(deg, x, wemb, bemb, wc0):
    return pl.pallas_call(
        _pre_body,
        grid=(N // BLK,),
        in_specs=[
            pl.BlockSpec((NC, BLK, HF), lambda i: (0, i, 0)),
            pl.BlockSpec((BLK, D), lambda i: (i, 0)),
            pl.BlockSpec((HF, D), lambda i: (0, 0)),
            pl.BlockSpec((1, HF), lambda i: (0, 0)),
            pl.BlockSpec((HF, HF), lambda i: (0, 0)),
        ],
        out_specs=[
            pl.BlockSpec((BLK, HF), lambda i: (i, 0)),
            pl.BlockSpec((BLK, 1), lambda i: (i, 0)),
        ],
        out_shape=[
            jax.ShapeDtypeStruct((N, HF), jnp.float32),
            jax.ShapeDtypeStruct((N, 1), jnp.float32),
        ],
    )(deg, x, wemb, bemb, wc0)


def _layer_body(acc_ref, g_ref, dis_ref, bc_ref, wc_ref, gout_ref):
    dis = dis_ref[...]
    h = jnp.maximum(
        dis * (acc_ref[0] + acc_ref[1] - g_ref[...]) + bc_ref[...], 0.0)
    gout_ref[...] = _mm_t(h, wc_ref[...]) * dis


def _layer_kernel(acc, g, dis, bc, wc):
    return pl.pallas_call(
        _layer_body,
        grid=(N // BLK,),
        in_specs=[
            pl.BlockSpec((NC, BLK, HF), lambda i: (0, i, 0)),
            pl.BlockSpec((BLK, HF), lambda i: (i, 0)),
            pl.BlockSpec((BLK, 1), lambda i: (i, 0)),
            pl.BlockSpec((1, HF), lambda i: (0, 0)),
            pl.BlockSpec((HF, HF), lambda i: (0, 0)),
        ],
        out_specs=pl.BlockSpec((BLK, HF), lambda i: (i, 0)),
        out_shape=jax.ShapeDtypeStruct((N, HF), jnp.float32),
    )(acc, g, dis, bc, wc)


def _head_body(acc_ref, g_ref, dis_ref, bc_ref, gf_ref,
               wg_ref, bg_ref, w1_ref, b1_ref, w2_ref, b2_ref,
               out_ref, sum_ref):
    i = pl.program_id(0)

    @pl.when(i == 0)
    def _():
        sum_ref[...] = jnp.zeros_like(sum_ref)

    h = jnp.maximum(
        dis_ref[...] * (acc_ref[0] + acc_ref[1] - g_ref[...]) + bc_ref[...],
        0.0)
    sum_ref[...] += jnp.sum(h, axis=0, keepdims=True)

    @pl.when(i == N // BLK - 1)
    def _():
        mean = sum_ref[...] * (1.0 / N)
        xg = jnp.maximum(_mm_t(gf_ref[...], wg_ref[...]) + bg_ref[...], 0.0)
        comb = jnp.concatenate([mean, xg], axis=1)
        z = jnp.maximum(_mm_t(comb, w1_ref[...]) + b1_ref[...], 0.0)
        out_ref[...] = _mm_t(z, w2_ref[...]) + b2_ref[...]


def _head_kernel(acc, g, dis, bc, gf, wg, bg, w1, b1, w2, b2):
    return pl.pallas_call(
        _head_body,
        grid=(N // BLK,),
        in_specs=[
            pl.BlockSpec((NC, BLK, HF), lambda i: (0, i, 0)),
            pl.BlockSpec((BLK, HF), lambda i: (i, 0)),
            pl.BlockSpec((BLK, 1), lambda i: (i, 0)),
            pl.BlockSpec((1, HF), lambda i: (0, 0)),
            pl.BlockSpec((1, G), lambda i: (0, 0)),
            pl.BlockSpec((HF, G), lambda i: (0, 0)),
            pl.BlockSpec((1, HF), lambda i: (0, 0)),
            pl.BlockSpec((HF, 2 * HF), lambda i: (0, 0)),
            pl.BlockSpec((1, HF), lambda i: (0, 0)),
            pl.BlockSpec((OUT, HF), lambda i: (0, 0)),
            pl.BlockSpec((1, OUT), lambda i: (0, 0)),
        ],
        out_specs=pl.BlockSpec((1, OUT), lambda i: (0, 0)),
        out_shape=jax.ShapeDtypeStruct((1, OUT), jnp.float32),
        scratch_shapes=[pltpu.VMEM((1, HF), jnp.float32)],
    )(acc, g, dis, bc, gf, wg, bg, w1, b1, w2, b2)


@jax.jit
def kernel(x, edge_index, global_features, W_emb, b_emb,
           Wc0, bc0, Wc1, bc1, Wc2, bc2, Wg, bg, W1, b1, W2, b2):
    src = edge_index[0]
    dst = edge_index[1]

    ones_n = jnp.ones((N, HF), jnp.float32)
    dacc = _gs_kernel()(ones_n, src, dst)

    g, dis = _pre_kernel(dacc, x, W_emb, b_emb.reshape(1, HF), Wc0)

    acc = _gs_kernel()(g, src, dst)
    g = _layer_kernel(acc, g, dis, bc0.reshape(1, HF), Wc1)

    acc = _gs_kernel()(g, src, dst)
    g = _layer_kernel(acc, g, dis, bc1.reshape(1, HF), Wc2)

    acc = _gs_kernel()(g, src, dst)
    out = _head_kernel(acc, g, dis, bc2.reshape(1, HF),
                       global_features.reshape(1, G),
                       Wg, bg.reshape(1, HF),
                       W1, b1.reshape(1, HF),
                       W2, b2.reshape(1, OUT))
    return out.reshape(OUT)


# speedup vs baseline: 18.8078x; 1.0871x over previous
import functools

import jax
import jax.numpy as jnp
from jax import lax
from jax.experimental import pallas as pl
from jax.experimental.pallas import tpu as pltpu
from jax.experimental.pallas import tpu_sc as plsc

N = 10000
E = 320000
D = 128
HF = 128
G = 32
OUT = 64

NC = 2
NS = 16
NW = NC * NS
EPT = E // NW
CHUNK = 128
NFULL = EPT // CHUNK
TAIL = EPT - NFULL * CHUNK
ROWS_PT = 624
ROWS_TL = N - NS * ROWS_PT
RCHUNK = 104
@functools.cache
def _mesh():
    return plsc.VectorSubcoreMesh(core_axis_name="c", subcore_axis_name="s",
                                  num_cores=NC, num_subcores=NS)


def _gs_body(g_hbm, src_hbm, dst_hbm, out_hbm,
             sidx0_v, didx0_v, rows0_v, sidx1_v, didx1_v, rows1_v,
             sidx2_v, didx2_v, rows2_v, didx_t,
             acc_sh, gsem0, gsem1, gsem2, ssem0, ssem1, ssem2):
    c = lax.axis_index("c")
    s = lax.axis_index("s")
    wid = s * NC + c
    base = pl.multiple_of(wid * EPT, 8)
    r0 = pl.multiple_of(s * ROWS_PT, 8)

    for k in range(ROWS_PT // RCHUNK):
        rk = pl.multiple_of(r0 + k * RCHUNK, 8)
        pltpu.sync_copy(g_hbm.at[pl.ds(rk, RCHUNK)],
                        rows0_v.at[pl.ds(0, RCHUNK)])
        pltpu.sync_copy(rows0_v.at[pl.ds(0, RCHUNK)],
                        acc_sh.at[pl.ds(rk, RCHUNK)])

    @pl.when(s == 0)
    def _():
        pltpu.sync_copy(g_hbm.at[pl.ds(NS * ROWS_PT, ROWS_TL)],
                        rows0_v.at[pl.ds(0, ROWS_TL)])
        pltpu.sync_copy(rows0_v.at[pl.ds(0, ROWS_TL)],
                        acc_sh.at[pl.ds(NS * ROWS_PT, ROWS_TL)])

    plsc.subcore_barrier()

    sets = ((sidx0_v, didx0_v, rows0_v, gsem0, ssem0),
            (sidx1_v, didx1_v, rows1_v, gsem1, ssem1),
            (sidx2_v, didx2_v, rows2_v, gsem2, ssem2))

    def triple(i, carry):
        gwaits = []
        for k, (sidx, didx, rows, gsem, ssem) in enumerate(sets):
            off = pl.multiple_of(base + (3 * i + k) * CHUNK, 8)

            @pl.when(i > 0)
            def _(rows=rows, didx=didx, ssem=ssem):
                pltpu.make_async_copy(rows, acc_sh.at[didx], ssem).wait()

            pltpu.sync_copy(src_hbm.at[pl.ds(off, CHUNK)], sidx)
            pltpu.sync_copy(dst_hbm.at[pl.ds(off, CHUNK)], didx)
            gwaits.append(pltpu.async_copy(g_hbm.at[sidx], rows, gsem))

        for (sidx, didx, rows, gsem, ssem), gw in zip(sets, gwaits):
            gw.wait()
            pltpu.async_copy(rows, acc_sh.at[didx], ssem, add=True)
        return carry

    lax.fori_loop(0, NFULL // 3, triple, 0)
    for (sidx, didx, rows, gsem, ssem) in sets:
        pltpu.make_async_copy(rows, acc_sh.at[didx], ssem).wait()

    off = pl.multiple_of(base + NFULL * CHUNK, 8)
    pltpu.sync_copy(src_hbm.at[pl.ds(off, TAIL)], sidx0_v.at[pl.ds(0, TAIL)])
    pltpu.sync_copy(dst_hbm.at[pl.ds(off, TAIL)], didx_t)
    pltpu.async_copy(g_hbm.at[sidx0_v.at[pl.ds(0, TAIL)]],
                     rows0_v.at[pl.ds(0, TAIL)], gsem0).wait()
    pltpu.sync_copy(rows0_v.at[pl.ds(0, TAIL)], acc_sh.at[didx_t], add=True)

    plsc.subcore_barrier()
    for k in range(ROWS_PT // RCHUNK):
        rk = pl.multiple_of(r0 + k * RCHUNK, 8)
        pltpu.sync_copy(acc_sh.at[pl.ds(rk, RCHUNK)],
                        rows0_v.at[pl.ds(0, RCHUNK)])
        pltpu.sync_copy(rows0_v.at[pl.ds(0, RCHUNK)],
                        out_hbm.at[c, pl.ds(rk, RCHUNK)])

    @pl.when(s == 0)
    def _():
        pltpu.sync_copy(acc_sh.at[pl.ds(NS * ROWS_PT, ROWS_TL)],
                        rows0_v.at[pl.ds(0, ROWS_TL)])
        pltpu.sync_copy(rows0_v.at[pl.ds(0, ROWS_TL)],
                        out_hbm.at[c, pl.ds(NS * ROWS_PT, ROWS_TL)])


@functools.cache
def _gs_kernel():
    return pl.kernel(
        _gs_body,
        out_type=jax.ShapeDtypeStruct((NC, N, HF), jnp.float32),
        mesh=_mesh(),
        scratch_types=[
            pltpu.VMEM((CHUNK,), jnp.int32),
            pltpu.VMEM((CHUNK,), jnp.int32),
            pltpu.VMEM((CHUNK, HF), jnp.float32),
            pltpu.VMEM((CHUNK,), jnp.int32),
            pltpu.VMEM((CHUNK,), jnp.int32),
            pltpu.VMEM((CHUNK, HF), jnp.float32),
            pltpu.VMEM((CHUNK,), jnp.int32),
            pltpu.VMEM((CHUNK,), jnp.int32),
            pltpu.VMEM((CHUNK, HF), jnp.float32),
            pltpu.VMEM((TAIL,), jnp.int32),
            pltpu.VMEM_SHARED((N, HF), jnp.float32),
            pltpu.SemaphoreType.DMA,
            pltpu.SemaphoreType.DMA,
            pltpu.SemaphoreType.DMA,
            pltpu.SemaphoreType.DMA,
            pltpu.SemaphoreType.DMA,
            pltpu.SemaphoreType.DMA,
        ],
    )


BLK = 1000


def _mm_t(a, b):
    return lax.dot_general(a, b, (((1,), (1,)), ((), ())),
                           preferred_element_type=jnp.float32)


def _pre_body(deg_ref, x_ref, wemb_ref, bemb_ref, wc0_ref,
              g_ref, dis_ref):
    deg = deg_ref[0, :, 0] + deg_ref[1, :, 0] - 1.0
    dis = lax.rsqrt(deg)
    dis_ref[...] = dis[:, None]
    h0 = _mm_t(x_ref[...], wemb_ref[...]) + bemb_ref[...]
    g_ref[...] = _mm_t(h0, wc0_ref[...]) * dis[:, None]


def _pre_kernel(deg, x, wemb, bemb, wc0):
    return pl.pallas_call(
        _pre_body,
        grid=(N // BLK,),
        in_specs=[
            pl.BlockSpec((NC, BLK, HF), lambda i: (0, i, 0)),
            pl.BlockSpec((BLK, D), lambda i: (i, 0)),
            pl.BlockSpec((HF, D), lambda i: (0, 0)),
            pl.BlockSpec((1, HF), lambda i: (0, 0)),
            pl.BlockSpec((HF, HF), lambda i: (0, 0)),
        ],
        out_specs=[
            pl.BlockSpec((BLK, HF), lambda i: (i, 0)),
            pl.BlockSpec((BLK, 1), lambda i: (i, 0)),
        ],
        out_shape=[
            jax.ShapeDtypeStruct((N, HF), jnp.float32),
            jax.ShapeDtypeStruct((N, 1), jnp.float32),
        ],
    )(deg, x, wemb, bemb, wc0)


def _layer_body(acc_ref, g_ref, dis_ref, bc_ref, wc_ref, gout_ref):
    dis = dis_ref[...]
    h = jnp.maximum(
        dis * (acc_ref[0] + acc_ref[1] - g_ref[...]) + bc_ref[...], 0.0)
    gout_ref[...] = _mm_t(h, wc_ref[...]) * dis


def _layer_kernel(acc, g, dis, bc, wc):
    return pl.pallas_call(
        _layer_body,
        grid=(N // BLK,),
        in_specs=[
            pl.BlockSpec((NC, BLK, HF), lambda i: (0, i, 0)),
            pl.BlockSpec((BLK, HF), lambda i: (i, 0)),
            pl.BlockSpec((BLK, 1), lambda i: (i, 0)),
            pl.BlockSpec((1, HF), lambda i: (0, 0)),
            pl.BlockSpec((HF, HF), lambda i: (0, 0)),
        ],
        out_specs=pl.BlockSpec((BLK, HF), lambda i: (i, 0)),
        out_shape=jax.ShapeDtypeStruct((N, HF), jnp.float32),
    )(acc, g, dis, bc, wc)


def _head_body(acc_ref, g_ref, dis_ref, bc_ref, gf_ref,
               wg_ref, bg_ref, w1_ref, b1_ref, w2_ref, b2_ref,
               out_ref, sum_ref):
    i = pl.program_id(0)

    @pl.when(i == 0)
    def _():
        sum_ref[...] = jnp.zeros_like(sum_ref)

    h = jnp.maximum(
        dis_ref[...] * (acc_ref[0] + acc_ref[1] - g_ref[...]) + bc_ref[...],
        0.0)
    sum_ref[...] += jnp.sum(h, axis=0, keepdims=True)

    @pl.when(i == N // BLK - 1)
    def _():
        mean = sum_ref[...] * (1.0 / N)
        xg = jnp.maximum(_mm_t(gf_ref[...], wg_ref[...]) + bg_ref[...], 0.0)
        comb = jnp.concatenate([mean, xg], axis=1)
        z = jnp.maximum(_mm_t(comb, w1_ref[...]) + b1_ref[...], 0.0)
        out_ref[...] = _mm_t(z, w2_ref[...]) + b2_ref[...]


def _head_kernel(acc, g, dis, bc, gf, wg, bg, w1, b1, w2, b2):
    return pl.pallas_call(
        _head_body,
        grid=(N // BLK,),
        in_specs=[
            pl.BlockSpec((NC, BLK, HF), lambda i: (0, i, 0)),
            pl.BlockSpec((BLK, HF), lambda i: (i, 0)),
            pl.BlockSpec((BLK, 1), lambda i: (i, 0)),
            pl.BlockSpec((1, HF), lambda i: (0, 0)),
            pl.BlockSpec((1, G), lambda i: (0, 0)),
            pl.BlockSpec((HF, G), lambda i: (0, 0)),
            pl.BlockSpec((1, HF), lambda i: (0, 0)),
            pl.BlockSpec((HF, 2 * HF), lambda i: (0, 0)),
            pl.BlockSpec((1, HF), lambda i: (0, 0)),
            pl.BlockSpec((OUT, HF), lambda i: (0, 0)),
            pl.BlockSpec((1, OUT), lambda i: (0, 0)),
        ],
        out_specs=pl.BlockSpec((1, OUT), lambda i: (0, 0)),
        out_shape=jax.ShapeDtypeStruct((1, OUT), jnp.float32),
        scratch_shapes=[pltpu.VMEM((1, HF), jnp.float32)],
    )(acc, g, dis, bc, gf, wg, bg, w1, b1, w2, b2)


@jax.jit
def kernel(x, edge_index, global_features, W_emb, b_emb,
           Wc0, bc0, Wc1, bc1, Wc2, bc2, Wg, bg, W1, b1, W2, b2):
    src = edge_index[0]
    dst = edge_index[1]

    ones_n = jnp.ones((N, HF), jnp.float32)
    dacc = _gs_kernel()(ones_n, src, dst)

    g, dis = _pre_kernel(dacc, x, W_emb, b_emb.reshape(1, HF), Wc0)

    acc = _gs_kernel()(g, src, dst)
    g = _layer_kernel(acc, g, dis, bc0.reshape(1, HF), Wc1)

    acc = _gs_kernel()(g, src, dst)
    g = _layer_kernel(acc, g, dis, bc1.reshape(1, HF), Wc2)

    acc = _gs_kernel()(g, src, dst)
    out = _head_kernel(acc, g, dis, bc2.reshape(1, HF),
                       global_features.reshape(1, G),
                       Wg, bg.reshape(1, HF),
                       W1, b1.reshape(1, HF),
                       W2, b2.reshape(1, OUT))
    return out.reshape(OUT)


# speedup vs baseline: 18.8689x; 1.0032x over previous
import functools

import jax
import jax.numpy as jnp
from jax import lax
from jax.experimental import pallas as pl
from jax.experimental.pallas import tpu as pltpu
from jax.experimental.pallas import tpu_sc as plsc

N = 10000
E = 320000
D = 128
HF = 128
G = 32
OUT = 64

NC = 2
NS = 16
NW = NC * NS
EPT = E // NW
CHUNK = 128
NFULL = EPT // CHUNK
TAIL = EPT - NFULL * CHUNK
ROWS_PT = 624
ROWS_TL = N - NS * ROWS_PT
RCHUNK = 104
@functools.cache
def _mesh():
    return plsc.VectorSubcoreMesh(core_axis_name="c", subcore_axis_name="s",
                                  num_cores=NC, num_subcores=NS)


def _gs_body(g_hbm, src_hbm, dst_hbm, out_hbm,
             sidx0_v, didx0_v, rows0_v, sidx1_v, didx1_v, rows1_v,
             sidx2_v, didx2_v, rows2_v, didx_t,
             acc_sh, gsem0, gsem1, gsem2, ssem0, ssem1, ssem2):
    c = lax.axis_index("c")
    s = lax.axis_index("s")
    wid = s * NC + c
    base = pl.multiple_of(wid * EPT, 8)
    r0 = pl.multiple_of(s * ROWS_PT, 8)

    for k in range(ROWS_PT // RCHUNK):
        rk = pl.multiple_of(r0 + k * RCHUNK, 8)
        pltpu.sync_copy(g_hbm.at[pl.ds(rk, RCHUNK)],
                        rows0_v.at[pl.ds(0, RCHUNK)])
        pltpu.sync_copy(rows0_v.at[pl.ds(0, RCHUNK)],
                        acc_sh.at[pl.ds(rk, RCHUNK)])

    @pl.when(s == 0)
    def _():
        pltpu.sync_copy(g_hbm.at[pl.ds(NS * ROWS_PT, ROWS_TL)],
                        rows0_v.at[pl.ds(0, ROWS_TL)])
        pltpu.sync_copy(rows0_v.at[pl.ds(0, ROWS_TL)],
                        acc_sh.at[pl.ds(NS * ROWS_PT, ROWS_TL)])

    plsc.subcore_barrier()

    sets = ((sidx0_v, didx0_v, rows0_v, gsem0, ssem0),
            (sidx1_v, didx1_v, rows1_v, gsem1, ssem1),
            (sidx2_v, didx2_v, rows2_v, gsem2, ssem2))

    def triple(i, carry):
        gwaits = []
        for k, (sidx, didx, rows, gsem, ssem) in enumerate(sets):
            off = pl.multiple_of(base + (3 * i + k) * CHUNK, 8)

            @pl.when(i > 0)
            def _(rows=rows, didx=didx, ssem=ssem):
                pltpu.make_async_copy(rows, acc_sh.at[didx], ssem).wait()

            pltpu.sync_copy(src_hbm.at[pl.ds(off, CHUNK)], sidx)
            pltpu.sync_copy(dst_hbm.at[pl.ds(off, CHUNK)], didx)
            gwaits.append(pltpu.async_copy(g_hbm.at[sidx], rows, gsem))

        for (sidx, didx, rows, gsem, ssem), gw in zip(sets, gwaits):
            gw.wait()
            pltpu.async_copy(rows, acc_sh.at[didx], ssem, add=True)
        return carry

    lax.fori_loop(0, NFULL // 3, triple, 0)
    for (sidx, didx, rows, gsem, ssem) in sets:
        pltpu.make_async_copy(rows, acc_sh.at[didx], ssem).wait()

    off = pl.multiple_of(base + NFULL * CHUNK, 8)
    pltpu.sync_copy(src_hbm.at[pl.ds(off, TAIL)], sidx0_v.at[pl.ds(0, TAIL)])
    pltpu.sync_copy(dst_hbm.at[pl.ds(off, TAIL)], didx_t)
    pltpu.async_copy(g_hbm.at[sidx0_v.at[pl.ds(0, TAIL)]],
                     rows0_v.at[pl.ds(0, TAIL)], gsem0).wait()
    pltpu.sync_copy(rows0_v.at[pl.ds(0, TAIL)], acc_sh.at[didx_t], add=True)

    plsc.subcore_barrier()
    for k in range(ROWS_PT // RCHUNK):
        rk = pl.multiple_of(r0 + k * RCHUNK, 8)
        pltpu.sync_copy(acc_sh.at[pl.ds(rk, RCHUNK)],
                        rows0_v.at[pl.ds(0, RCHUNK)])
        pltpu.sync_copy(rows0_v.at[pl.ds(0, RCHUNK)],
                        out_hbm.at[c, pl.ds(rk, RCHUNK)])

    @pl.when(s == 0)
    def _():
        pltpu.sync_copy(acc_sh.at[pl.ds(NS * ROWS_PT, ROWS_TL)],
                        rows0_v.at[pl.ds(0, ROWS_TL)])
        pltpu.sync_copy(rows0_v.at[pl.ds(0, ROWS_TL)],
                        out_hbm.at[c, pl.ds(NS * ROWS_PT, ROWS_TL)])


@functools.cache
def _gs_kernel():
    return pl.kernel(
        _gs_body,
        out_type=jax.ShapeDtypeStruct((NC, N, HF), jnp.float32),
        mesh=_mesh(),
        scratch_types=[
            pltpu.VMEM((CHUNK,), jnp.int32),
            pltpu.VMEM((CHUNK,), jnp.int32),
            pltpu.VMEM((CHUNK, HF), jnp.float32),
            pltpu.VMEM((CHUNK,), jnp.int32),
            pltpu.VMEM((CHUNK,), jnp.int32),
            pltpu.VMEM((CHUNK, HF), jnp.float32),
            pltpu.VMEM((CHUNK,), jnp.int32),
            pltpu.VMEM((CHUNK,), jnp.int32),
            pltpu.VMEM((CHUNK, HF), jnp.float32),
            pltpu.VMEM((TAIL,), jnp.int32),
            pltpu.VMEM_SHARED((N, HF), jnp.float32),
            pltpu.SemaphoreType.DMA,
            pltpu.SemaphoreType.DMA,
            pltpu.SemaphoreType.DMA,
            pltpu.SemaphoreType.DMA,
            pltpu.SemaphoreType.DMA,
            pltpu.SemaphoreType.DMA,
        ],
    )


BLK = 1000


def _mm_t(a, b):
    return lax.dot_general(a, b, (((1,), (1,)), ((), ())),
                           preferred_element_type=jnp.float32)


def _mm0_body(x_ref, wemb_ref, bemb_ref, wc0_ref, out_ref):
    h0 = _mm_t(x_ref[...], wemb_ref[...]) + bemb_ref[...]
    out_ref[...] = _mm_t(h0, wc0_ref[...])


def _mm0_kernel(x, wemb, bemb, wc0):
    return pl.pallas_call(
        _mm0_body,
        grid=(N // BLK,),
        in_specs=[
            pl.BlockSpec((BLK, D), lambda i: (i, 0)),
            pl.BlockSpec((HF, D), lambda i: (0, 0)),
            pl.BlockSpec((1, HF), lambda i: (0, 0)),
            pl.BlockSpec((HF, HF), lambda i: (0, 0)),
        ],
        out_specs=pl.BlockSpec((BLK, HF), lambda i: (i, 0)),
        out_shape=jax.ShapeDtypeStruct((N, HF), jnp.float32),
    )(x, wemb, bemb, wc0)


def _pre_body(deg_ref, hw_ref, g_ref, dis_ref):
    deg = deg_ref[0, :, 0] + deg_ref[1, :, 0] - 1.0
    dis = lax.rsqrt(deg)
    dis_ref[...] = dis[:, None]
    g_ref[...] = hw_ref[...] * dis[:, None]


def _pre_kernel(deg, hw):
    return pl.pallas_call(
        _pre_body,
        grid=(N // BLK,),
        in_specs=[
            pl.BlockSpec((NC, BLK, HF), lambda i: (0, i, 0)),
            pl.BlockSpec((BLK, HF), lambda i: (i, 0)),
        ],
        out_specs=[
            pl.BlockSpec((BLK, HF), lambda i: (i, 0)),
            pl.BlockSpec((BLK, 1), lambda i: (i, 0)),
        ],
        out_shape=[
            jax.ShapeDtypeStruct((N, HF), jnp.float32),
            jax.ShapeDtypeStruct((N, 1), jnp.float32),
        ],
    )(deg, hw)


def _layer_body(acc_ref, g_ref, dis_ref, bc_ref, wc_ref, gout_ref):
    dis = dis_ref[...]
    h = jnp.maximum(
        dis * (acc_ref[0] + acc_ref[1] - g_ref[...]) + bc_ref[...], 0.0)
    gout_ref[...] = _mm_t(h, wc_ref[...]) * dis


def _layer_kernel(acc, g, dis, bc, wc):
    return pl.pallas_call(
        _layer_body,
        grid=(N // BLK,),
        in_specs=[
            pl.BlockSpec((NC, BLK, HF), lambda i: (0, i, 0)),
            pl.BlockSpec((BLK, HF), lambda i: (i, 0)),
            pl.BlockSpec((BLK, 1), lambda i: (i, 0)),
            pl.BlockSpec((1, HF), lambda i: (0, 0)),
            pl.BlockSpec((HF, HF), lambda i: (0, 0)),
        ],
        out_specs=pl.BlockSpec((BLK, HF), lambda i: (i, 0)),
        out_shape=jax.ShapeDtypeStruct((N, HF), jnp.float32),
    )(acc, g, dis, bc, wc)


def _head_body(acc_ref, g_ref, dis_ref, bc_ref, gf_ref,
               wg_ref, bg_ref, w1_ref, b1_ref, w2_ref, b2_ref,
               out_ref, sum_ref):
    i = pl.program_id(0)

    @pl.when(i == 0)
    def _():
        sum_ref[...] = jnp.zeros_like(sum_ref)

    h = jnp.maximum(
        dis_ref[...] * (acc_ref[0] + acc_ref[1] - g_ref[...]) + bc_ref[...],
        0.0)
    sum_ref[...] += jnp.sum(h, axis=0, keepdims=True)

    @pl.when(i == N // BLK - 1)
    def _():
        mean = sum_ref[...] * (1.0 / N)
        xg = jnp.maximum(_mm_t(gf_ref[...], wg_ref[...]) + bg_ref[...], 0.0)
        comb = jnp.concatenate([mean, xg], axis=1)
        z = jnp.maximum(_mm_t(comb, w1_ref[...]) + b1_ref[...], 0.0)
        out_ref[...] = _mm_t(z, w2_ref[...]) + b2_ref[...]


def _head_kernel(acc, g, dis, bc, gf, wg, bg, w1, b1, w2, b2):
    return pl.pallas_call(
        _head_body,
        grid=(N // BLK,),
        in_specs=[
            pl.BlockSpec((NC, BLK, HF), lambda i: (0, i, 0)),
            pl.BlockSpec((BLK, HF), lambda i: (i, 0)),
            pl.BlockSpec((BLK, 1), lambda i: (i, 0)),
            pl.BlockSpec((1, HF), lambda i: (0, 0)),
            pl.BlockSpec((1, G), lambda i: (0, 0)),
            pl.BlockSpec((HF, G), lambda i: (0, 0)),
            pl.BlockSpec((1, HF), lambda i: (0, 0)),
            pl.BlockSpec((HF, 2 * HF), lambda i: (0, 0)),
            pl.BlockSpec((1, HF), lambda i: (0, 0)),
            pl.BlockSpec((OUT, HF), lambda i: (0, 0)),
            pl.BlockSpec((1, OUT), lambda i: (0, 0)),
        ],
        out_specs=pl.BlockSpec((1, OUT), lambda i: (0, 0)),
        out_shape=jax.ShapeDtypeStruct((1, OUT), jnp.float32),
        scratch_shapes=[pltpu.VMEM((1, HF), jnp.float32)],
    )(acc, g, dis, bc, gf, wg, bg, w1, b1, w2, b2)


@jax.jit
def kernel(x, edge_index, global_features, W_emb, b_emb,
           Wc0, bc0, Wc1, bc1, Wc2, bc2, Wg, bg, W1, b1, W2, b2):
    src = edge_index[0]
    dst = edge_index[1]

    ones_n = jnp.ones((N, HF), jnp.float32)
    hw0 = _mm0_kernel(x, W_emb, b_emb.reshape(1, HF), Wc0)
    dacc = _gs_kernel()(ones_n, src, dst)
    g, dis = _pre_kernel(dacc, hw0)

    acc = _gs_kernel()(g, src, dst)
    g = _layer_kernel(acc, g, dis, bc0.reshape(1, HF), Wc1)

    acc = _gs_kernel()(g, src, dst)
    g = _layer_kernel(acc, g, dis, bc1.reshape(1, HF), Wc2)

    acc = _gs_kernel()(g, src, dst)
    out = _head_kernel(acc, g, dis, bc2.reshape(1, HF),
                       global_features.reshape(1, G),
                       Wg, bg.reshape(1, HF),
                       W1, b1.reshape(1, HF),
                       W2, b2.reshape(1, OUT))
    return out.reshape(OUT)
